# baseline (speedup 1.0000x reference)
import functools

import jax
import jax.numpy as jnp
from jax import lax
from jax.experimental import pallas as pl
from jax.experimental.pallas import tpu as pltpu
from jax.experimental.pallas import tpu_sc as plsc

NBINS = 256
START = -15.0
STEP = 30.0 / 256.0
INV_STEP = 256.0 / 30.0
B, N, C = 1024, 100, 4
LANES = 16
BLK = 128


def _kde_body(rows_per_w, vT_hbm, wT_hbm, out, vv, wv, acc, sem1, sem2):
    nc = 2
    wid = lax.axis_index("s") * nc + lax.axis_index("c")
    r0 = wid * rows_per_w
    c0 = (wid // 4) * BLK
    sub = (wid % 4) * rows_per_w

    cp1 = pltpu.async_copy(vT_hbm.at[:, pl.ds(c0, BLK)], vv, sem1)
    cp2 = pltpu.async_copy(wT_hbm.at[:, pl.ds(c0, BLK)], wv, sem2)

    zeros = jnp.zeros((LANES,), jnp.float32)

    def zbody(r, _):
        for u in range(NBINS * C // LANES):
            acc[r, pl.ds(u * LANES, LANES)] = zeros
        return 0

    lax.fori_loop(0, rows_per_w, zbody, 0)

    cp1.wait()
    cp2.wait()

    iota = lax.iota(jnp.int32, LANES)
    for g in range(rows_per_w // LANES):
        rows = g * LANES + iota

        def nbody(n, _, g=g, rows=rows):
            v = vv[n, pl.ds(sub + g * LANES, LANES)]
            t = (v - START) * INV_STEP
            j = t.astype(jnp.int32)
            e_lo = j.astype(jnp.float32) * STEP + START
            j = j - jnp.where(v < e_lo, 1, 0) + jnp.where(v >= e_lo + STEP, 1, 0)
            j = jnp.minimum(jnp.maximum(j, 0), NBINS - 1)
            jc = j * C
            for c in range(C):
                w = wv[n * C + c, pl.ds(sub + g * LANES, LANES)]
                plsc.addupdate_scatter(acc, [rows, jc + c], w)
            return 0

        lax.fori_loop(0, N, nbody, 0)

    pltpu.sync_copy(acc, out.at[pl.ds(r0, rows_per_w), :])


def kernel(value, weights):
    mesh = plsc.VectorSubcoreMesh(core_axis_name="c", subcore_axis_name="s")
    nworkers = mesh.num_cores * mesh.num_subcores
    rows_per_w = B // nworkers

    vT = value.T
    wT = weights.reshape(B, N * C).T

    run = pl.kernel(
        functools.partial(_kde_body, rows_per_w),
        out_type=jax.ShapeDtypeStruct((B, NBINS * C), jnp.float32),
        mesh=mesh,
        compiler_params=pltpu.CompilerParams(needs_layout_passes=False),
        scratch_types=[
            pltpu.VMEM((N, BLK), jnp.float32),
            pltpu.VMEM((N * C, BLK), jnp.float32),
            pltpu.VMEM((rows_per_w, NBINS * C), jnp.float32),
            pltpu.SemaphoreType.DMA,
            pltpu.SemaphoreType.DMA,
        ],
    )
    h = run(vT, wT)
    return h.reshape(B, NBINS, C)


# speedup vs baseline: 1.1841x; 1.1841x over previous
import functools

import jax
import jax.numpy as jnp
from jax import lax
from jax.experimental import pallas as pl
from jax.experimental.pallas import tpu as pltpu
from jax.experimental.pallas import tpu_sc as plsc

NBINS = 256
START = -15.0
STEP = 30.0 / 256.0
INV_STEP = 256.0 / 30.0
B, N, C = 1024, 100, 4
LANES = 16
BLK = 128


NSPLIT = 48


def _kde_body(rows_per_w, vT_hbm, wT_hbm, o0, o1, o2, o3,
              vv, wv, a0, a1, a2, a3, sem1, sem2, sem3, semo):
    nc = 2
    wid = lax.axis_index("s") * nc + lax.axis_index("c")
    r0 = wid * rows_per_w
    c0 = (wid // 4) * BLK
    sub = (wid % 4) * rows_per_w
    outs = (o0, o1, o2, o3)
    accs = (a0, a1, a2, a3)

    cp1 = pltpu.async_copy(vT_hbm.at[:, pl.ds(c0, BLK)], vv, sem1)
    cp2 = pltpu.async_copy(wT_hbm.at[pl.ds(0, NSPLIT * C), pl.ds(c0, BLK)],
                           wv.at[pl.ds(0, NSPLIT * C)], sem2)
    cp3 = pltpu.async_copy(wT_hbm.at[pl.ds(NSPLIT * C, (N - NSPLIT) * C), pl.ds(c0, BLK)],
                           wv.at[pl.ds(NSPLIT * C, (N - NSPLIT) * C)], sem3)

    zeros = jnp.zeros((LANES,), jnp.float32)

    def zbody(r, _):
        for acc in accs:
            for u in range(NBINS // LANES):
                acc[r, pl.ds(u * LANES, LANES)] = zeros
        return 0

    lax.fori_loop(0, rows_per_w, zbody, 0)

    cp1.wait()
    cp2.wait()

    iota = lax.iota(jnp.int32, LANES)

    def make_nbody(g):
        rows = g * LANES + iota

        def nbody(n, _):
            v = vv[n, pl.ds(sub + g * LANES, LANES)]
            t = (v - START) * INV_STEP
            j = t.astype(jnp.int32)
            e_lo = j.astype(jnp.float32) * STEP + START
            j = j - jnp.where(v < e_lo, 1, 0) + jnp.where(v >= e_lo + STEP, 1, 0)
            j = jnp.minimum(jnp.maximum(j, 0), NBINS - 1)
            for c in range(C):
                w = wv[n * C + c, pl.ds(sub + g * LANES, LANES)]
                plsc.addupdate_scatter(accs[c], [rows, j], w)
            return 0

        return nbody

    lax.fori_loop(0, NSPLIT, make_nbody(0), 0)
    cp3.wait()
    lax.fori_loop(NSPLIT, N, make_nbody(0), 0)

    cpo = [pltpu.async_copy(accs[c].at[pl.ds(0, LANES)],
                            outs[c].at[pl.ds(r0, LANES), :], semo)
           for c in range(C)]

    lax.fori_loop(0, N, make_nbody(1), 0)

    for cp in cpo:
        cp.wait()
    for c in range(C):
        pltpu.sync_copy(accs[c].at[pl.ds(LANES, LANES)],
                        outs[c].at[pl.ds(r0 + LANES, LANES), :])


def kernel(value, weights):
    mesh = plsc.VectorSubcoreMesh(core_axis_name="c", subcore_axis_name="s")
    nworkers = mesh.num_cores * mesh.num_subcores
    rows_per_w = B // nworkers

    vT = value.T
    wT = weights.reshape(B, N * C).T

    run = pl.kernel(
        functools.partial(_kde_body, rows_per_w),
        out_type=[jax.ShapeDtypeStruct((B, NBINS), jnp.float32)] * C,
        mesh=mesh,
        compiler_params=pltpu.CompilerParams(needs_layout_passes=False),
        scratch_types=[
            pltpu.VMEM((N, BLK), jnp.float32),
            pltpu.VMEM((N * C, BLK), jnp.float32),
        ] + [pltpu.VMEM((rows_per_w, NBINS), jnp.float32)] * C + [
            pltpu.SemaphoreType.DMA,
            pltpu.SemaphoreType.DMA,
            pltpu.SemaphoreType.DMA,
            pltpu.SemaphoreType.DMA,
        ],
    )
    h0, h1, h2, h3 = run(vT, wT)
    return jnp.stack([h0, h1, h2, h3], axis=2)


# speedup vs baseline: 1.3101x; 1.1065x over previous
import functools

import jax
import jax.numpy as jnp
from jax import lax
from jax.experimental import pallas as pl
from jax.experimental.pallas import tpu as pltpu
from jax.experimental.pallas import tpu_sc as plsc

NBINS = 256
START = -15.0
STEP = 30.0 / 256.0
INV_STEP = 256.0 / 30.0
B, N, C = 1024, 100, 4
LANES = 16
BLK = 128


NSPLIT = 48


def _kde_body(rows_per_w, vT_hbm, wT_hbm, o0, o1, o2, o3,
              vv, wv, a0, a1, a2, a3, sem1, sem2, sem3, semo):
    nc = 2
    wid = lax.axis_index("s") * nc + lax.axis_index("c")
    r0 = wid * rows_per_w
    c0 = (wid // 4) * BLK
    sub = (wid % 4) * rows_per_w
    outs = (o0, o1, o2, o3)
    accs = (a0, a1, a2, a3)

    cp1 = pltpu.async_copy(vT_hbm.at[:, pl.ds(c0, BLK)], vv, sem1)
    cp2 = pltpu.async_copy(wT_hbm.at[pl.ds(0, NSPLIT * C), pl.ds(c0, BLK)],
                           wv.at[pl.ds(0, NSPLIT * C)], sem2)
    cp3 = pltpu.async_copy(wT_hbm.at[pl.ds(NSPLIT * C, (N - NSPLIT) * C), pl.ds(c0, BLK)],
                           wv.at[pl.ds(NSPLIT * C, (N - NSPLIT) * C)], sem3)

    zeros = jnp.zeros((LANES,), jnp.float32)

    def zbody(r, _):
        for acc in accs:
            for u in range(NBINS // LANES):
                acc[r, pl.ds(u * LANES, LANES)] = zeros
        return 0

    lax.fori_loop(0, rows_per_w, zbody, 0)

    cp1.wait()
    cp2.wait()

    iota = lax.iota(jnp.int32, LANES)

    def bin_of(v):
        t = (v - START) * INV_STEP
        j = t.astype(jnp.int32)
        e_lo = j.astype(jnp.float32) * STEP + START
        j = j - jnp.where(v < e_lo, 1, 0) + jnp.where(v >= e_lo + STEP, 1, 0)
        return jnp.minimum(jnp.maximum(j, 0), NBINS - 1)

    def make_nbody(g, unroll):
        rows = g * LANES + iota
        lane0 = sub + g * LANES

        def nbody(i, _):
            n = i * unroll
            vs = [vv[n + u, pl.ds(lane0, LANES)] for u in range(unroll)]
            ws = [[wv[(n + u) * C + c, pl.ds(lane0, LANES)] for c in range(C)]
                  for u in range(unroll)]
            js = [bin_of(v) for v in vs]
            for u in range(unroll):
                for c in range(C):
                    plsc.addupdate_scatter(accs[c], [rows, js[u]], ws[u][c])
            return 0

        return nbody

    lax.fori_loop(0, NSPLIT // 2, make_nbody(0, 2), 0)
    cp3.wait()
    lax.fori_loop(NSPLIT // 2, N // 2, make_nbody(0, 2), 0)

    cpo = [pltpu.async_copy(accs[c].at[pl.ds(0, LANES)],
                            outs[c].at[pl.ds(r0, LANES), :], semo)
           for c in range(C)]

    lax.fori_loop(0, N // 2, make_nbody(1, 2), 0)

    for cp in cpo:
        cp.wait()
    for c in range(C):
        pltpu.sync_copy(accs[c].at[pl.ds(LANES, LANES)],
                        outs[c].at[pl.ds(r0 + LANES, LANES), :])


def kernel(value, weights):
    mesh = plsc.VectorSubcoreMesh(core_axis_name="c", subcore_axis_name="s")
    nworkers = mesh.num_cores * mesh.num_subcores
    rows_per_w = B // nworkers

    vT = value.T
    wT = weights.reshape(B, N * C).T

    run = pl.kernel(
        functools.partial(_kde_body, rows_per_w),
        out_type=[jax.ShapeDtypeStruct((B, NBINS), jnp.float32)] * C,
        mesh=mesh,
        compiler_params=pltpu.CompilerParams(needs_layout_passes=False),
        scratch_types=[
            pltpu.VMEM((N, BLK), jnp.float32),
            pltpu.VMEM((N * C, BLK), jnp.float32),
        ] + [pltpu.VMEM((rows_per_w, NBINS), jnp.float32)] * C + [
            pltpu.SemaphoreType.DMA,
            pltpu.SemaphoreType.DMA,
            pltpu.SemaphoreType.DMA,
            pltpu.SemaphoreType.DMA,
        ],
    )
    h0, h1, h2, h3 = run(vT, wT)
    return jnp.stack([h0, h1, h2, h3], axis=2)


# speedup vs baseline: 1.3582x; 1.0367x over previous
import functools

import jax
import jax.numpy as jnp
from jax import lax
from jax.experimental import pallas as pl
from jax.experimental.pallas import tpu as pltpu
from jax.experimental.pallas import tpu_sc as plsc

NBINS = 256
START = -15.0
STEP = 30.0 / 256.0
INV_STEP = 256.0 / 30.0
B, N, C = 1024, 100, 4
LANES = 16
BLK = 128


NSPLIT = 48


def _kde_body(rows_per_w, vT_hbm, wT_hbm, o0, o1, o2, o3,
              vv, wv, a0, a1, a2, a3, sem1, sem2, sem3, semo):
    nc = 2
    wid = lax.axis_index("s") * nc + lax.axis_index("c")
    r0 = wid * rows_per_w
    c0 = (wid // 4) * BLK
    sub = (wid % 4) * rows_per_w
    outs = (o0, o1, o2, o3)
    accs = (a0, a1, a2, a3)

    cp1 = pltpu.async_copy(vT_hbm.at[:, pl.ds(c0, BLK)], vv, sem1)
    cp2 = pltpu.async_copy(wT_hbm.at[pl.ds(0, NSPLIT * C), pl.ds(c0, BLK)],
                           wv.at[pl.ds(0, NSPLIT * C)], sem2)
    cp3 = pltpu.async_copy(wT_hbm.at[pl.ds(NSPLIT * C, (N - NSPLIT) * C), pl.ds(c0, BLK)],
                           wv.at[pl.ds(NSPLIT * C, (N - NSPLIT) * C)], sem3)

    zeros = jnp.zeros((LANES,), jnp.float32)

    def zbody(r, _):
        for acc in accs:
            for u in range(NBINS // LANES):
                acc[r, pl.ds(u * LANES, LANES)] = zeros
        return 0

    lax.fori_loop(0, rows_per_w, zbody, 0)

    cp1.wait()
    cp2.wait()

    iota = lax.iota(jnp.int32, LANES)

    def bin_of(v):
        t = (v - START) * INV_STEP
        j = t.astype(jnp.int32)
        e_lo = j.astype(jnp.float32) * STEP + START
        j = j - jnp.where(v < e_lo, 1, 0) + jnp.where(v >= e_lo + STEP, 1, 0)
        return jnp.minimum(jnp.maximum(j, 0), NBINS - 1)

    def make_nbody(g, unroll):
        rows = g * LANES + iota
        lane0 = sub + g * LANES

        def nbody(i, _):
            n = i * unroll
            vs = [vv[n + u, pl.ds(lane0, LANES)] for u in range(unroll)]
            ws = [[wv[(n + u) * C + c, pl.ds(lane0, LANES)] for c in range(C)]
                  for u in range(unroll)]
            js = [bin_of(v) for v in vs]
            for u in range(unroll):
                for c in range(C):
                    plsc.addupdate_scatter(accs[c], [rows, js[u]], ws[u][c])
            return 0

        return nbody

    lax.fori_loop(0, NSPLIT // 4, make_nbody(0, 4), 0)
    cp3.wait()
    lax.fori_loop(NSPLIT // 4, N // 4, make_nbody(0, 4), 0)

    cpo = [pltpu.async_copy(accs[c].at[pl.ds(0, LANES)],
                            outs[c].at[pl.ds(r0, LANES), :], semo)
           for c in range(C)]

    lax.fori_loop(0, N // 4, make_nbody(1, 4), 0)

    for cp in cpo:
        cp.wait()
    for c in range(C):
        pltpu.sync_copy(accs[c].at[pl.ds(LANES, LANES)],
                        outs[c].at[pl.ds(r0 + LANES, LANES), :])


def kernel(value, weights):
    mesh = plsc.VectorSubcoreMesh(core_axis_name="c", subcore_axis_name="s")
    nworkers = mesh.num_cores * mesh.num_subcores
    rows_per_w = B // nworkers

    vT = value.T
    wT = weights.reshape(B, N * C).T

    run = pl.kernel(
        functools.partial(_kde_body, rows_per_w),
        out_type=[jax.ShapeDtypeStruct((B, NBINS), jnp.float32)] * C,
        mesh=mesh,
        compiler_params=pltpu.CompilerParams(needs_layout_passes=False),
        scratch_types=[
            pltpu.VMEM((N, BLK), jnp.float32),
            pltpu.VMEM((N * C, BLK), jnp.float32),
        ] + [pltpu.VMEM((rows_per_w, NBINS), jnp.float32)] * C + [
            pltpu.SemaphoreType.DMA,
            pltpu.SemaphoreType.DMA,
            pltpu.SemaphoreType.DMA,
            pltpu.SemaphoreType.DMA,
        ],
    )
    h0, h1, h2, h3 = run(vT, wT)
    return jnp.stack([h0, h1, h2, h3], axis=2)


# speedup vs baseline: 1.3605x; 1.0017x over previous
import functools

import jax
import jax.numpy as jnp
from jax import lax
from jax.experimental import pallas as pl
from jax.experimental.pallas import tpu as pltpu
from jax.experimental.pallas import tpu_sc as plsc

NBINS = 256
START = -15.0
STEP = 30.0 / 256.0
INV_STEP = 256.0 / 30.0
B, N, C = 1024, 100, 4
LANES = 16
BLK = 128


NSPLIT = 48


def _kde_body(rows_per_w, vT_hbm, wT_hbm, o0, o1, o2, o3,
              vv, wv, a0, a1, a2, a3, sem1, sem2, sem3, semo):
    nc = 2
    wid = lax.axis_index("s") * nc + lax.axis_index("c")
    r0 = wid * rows_per_w
    c0 = (wid // 4) * BLK
    sub = (wid % 4) * rows_per_w
    outs = (o0, o1, o2, o3)
    accs = (a0, a1, a2, a3)

    cp1 = pltpu.async_copy(vT_hbm.at[:, pl.ds(c0, BLK)], vv, sem1)
    cp2 = pltpu.async_copy(wT_hbm.at[pl.ds(0, NSPLIT * C), pl.ds(c0, BLK)],
                           wv.at[pl.ds(0, NSPLIT * C)], sem2)
    cp3 = pltpu.async_copy(wT_hbm.at[pl.ds(NSPLIT * C, (N - NSPLIT) * C), pl.ds(c0, BLK)],
                           wv.at[pl.ds(NSPLIT * C, (N - NSPLIT) * C)], sem3)

    zeros = jnp.zeros((LANES,), jnp.float32)

    def zbody(r, _):
        for acc in accs:
            for u in range(NBINS // LANES):
                acc[r, pl.ds(u * LANES, LANES)] = zeros
        return 0

    lax.fori_loop(0, rows_per_w, zbody, 0)

    cp1.wait()
    cp2.wait()

    iota = lax.iota(jnp.int32, LANES)

    def bin_of(v):
        t = (v - START) * INV_STEP
        j = t.astype(jnp.int32)
        e_lo = j.astype(jnp.float32) * STEP + START
        j = j - jnp.where(v < e_lo, 1, 0) + jnp.where(v >= e_lo + STEP, 1, 0)
        return jnp.minimum(jnp.maximum(j, 0), NBINS - 1)

    def make_nbody(g, unroll):
        rows = g * LANES + iota
        lane0 = sub + g * LANES

        def nbody(i, _):
            n = i * unroll
            vs = [vv[n + u, pl.ds(lane0, LANES)] for u in range(unroll)]
            ws = [[wv[(n + u) * C + c, pl.ds(lane0, LANES)] for c in range(C)]
                  for u in range(unroll)]
            js = [bin_of(v) for v in vs]
            for u in range(unroll):
                for c in range(C):
                    plsc.addupdate_scatter(accs[c], [rows, js[u]], ws[u][c])
            return 0

        return nbody

    lax.fori_loop(0, NSPLIT // 4, make_nbody(0, 4), 0)
    cp3.wait()
    lax.fori_loop(NSPLIT // 4, N // 4, make_nbody(0, 4), 0)

    cpo = [pltpu.async_copy(accs[c].at[pl.ds(0, LANES)],
                            outs[c].at[pl.ds(r0, LANES), :], semo)
           for c in range(C)]

    lax.fori_loop(0, N // 4, make_nbody(1, 4), 0)

    for cp in cpo:
        cp.wait()
    for c in range(C):
        pltpu.sync_copy(accs[c].at[pl.ds(LANES, LANES)],
                        outs[c].at[pl.ds(r0 + LANES, LANES), :])


def kernel(value, weights):
    mesh = plsc.VectorSubcoreMesh(core_axis_name="c", subcore_axis_name="s")
    nworkers = mesh.num_cores * mesh.num_subcores
    rows_per_w = B // nworkers

    vT = value.T
    wT = jnp.transpose(weights, (1, 2, 0)).reshape(N * C, B)

    run = pl.kernel(
        functools.partial(_kde_body, rows_per_w),
        out_type=[jax.ShapeDtypeStruct((B, NBINS), jnp.float32)] * C,
        mesh=mesh,
        compiler_params=pltpu.CompilerParams(needs_layout_passes=False),
        scratch_types=[
            pltpu.VMEM((N, BLK), jnp.float32),
            pltpu.VMEM((N * C, BLK), jnp.float32),
        ] + [pltpu.VMEM((rows_per_w, NBINS), jnp.float32)] * C + [
            pltpu.SemaphoreType.DMA,
            pltpu.SemaphoreType.DMA,
            pltpu.SemaphoreType.DMA,
            pltpu.SemaphoreType.DMA,
        ],
    )
    h0, h1, h2, h3 = run(vT, wT)
    return jnp.stack([h0, h1, h2, h3], axis=2)


# speedup vs baseline: 1.7285x; 1.2705x over previous
import functools

import jax
import jax.numpy as jnp
from jax import lax
from jax.experimental import pallas as pl
from jax.experimental.pallas import tpu as pltpu
from jax.experimental.pallas import tpu_sc as plsc

NBINS = 256
START = -15.0
STEP = 30.0 / 256.0
INV_STEP = 256.0 / 30.0
B, N, C = 1024, 100, 4
LANES = 16
BLK = 128


NSPLIT = 48


ROWW = NBINS * C


def _kde_body(rows_per_w, vT_hbm, wT_hbm, out, vv, wv, acc, sem1, sem2, sem3, semo):
    nc = 2
    wid = lax.axis_index("s") * nc + lax.axis_index("c")
    r0 = wid * rows_per_w
    c0 = (wid // 4) * BLK
    sub = (wid % 4) * rows_per_w

    cp1 = pltpu.async_copy(vT_hbm.at[:, pl.ds(c0, BLK)], vv, sem1)
    cp2 = pltpu.async_copy(wT_hbm.at[pl.ds(0, NSPLIT * C), pl.ds(c0, BLK)],
                           wv.at[pl.ds(0, NSPLIT * C)], sem2)
    cp3 = pltpu.async_copy(wT_hbm.at[pl.ds(NSPLIT * C, (N - NSPLIT) * C), pl.ds(c0, BLK)],
                           wv.at[pl.ds(NSPLIT * C, (N - NSPLIT) * C)], sem3)

    zeros = jnp.zeros((LANES,), jnp.float32)

    def zbody(i, _):
        base = i * (8 * LANES)
        for u in range(8):
            acc[pl.ds(base + u * LANES, LANES)] = zeros
        return 0

    lax.fori_loop(0, rows_per_w * ROWW // (8 * LANES), zbody, 0)

    cp1.wait()
    cp2.wait()

    iota = lax.iota(jnp.int32, LANES)

    def bin_of(v):
        t = (v - START) * INV_STEP
        j = t.astype(jnp.int32)
        e_lo = j.astype(jnp.float32) * STEP + START
        j = j - jnp.where(v < e_lo, 1, 0) + jnp.where(v >= e_lo + STEP, 1, 0)
        return jnp.minimum(jnp.maximum(j, 0), NBINS - 1)

    def make_nbody(g, unroll):
        rowbase = (g * LANES + iota) * ROWW
        lane0 = sub + g * LANES

        def nbody(i, _):
            n = i * unroll
            vs = [vv[n + u, pl.ds(lane0, LANES)] for u in range(unroll)]
            ws = [[wv[(n + u) * C + c, pl.ds(lane0, LANES)] for c in range(C)]
                  for u in range(unroll)]
            js = [bin_of(v) for v in vs]
            for u in range(unroll):
                j = js[u]
                base = rowbase + ((j & 128) << 2) + (j & 127)
                for c in range(C):
                    plsc.addupdate_scatter(acc, [base + c * 128], ws[u][c])
            return 0

        return nbody

    lax.fori_loop(0, NSPLIT // 4, make_nbody(0, 4), 0)
    cp3.wait()
    lax.fori_loop(NSPLIT // 4, N // 4, make_nbody(0, 4), 0)

    half = LANES * ROWW
    cpo = pltpu.async_copy(acc.at[pl.ds(0, half)],
                           out.at[pl.ds(r0 * ROWW, half)], semo)

    lax.fori_loop(0, N // 4, make_nbody(1, 4), 0)

    cpo.wait()
    pltpu.sync_copy(acc.at[pl.ds(half, half)],
                    out.at[pl.ds(r0 * ROWW + half, half)])


def kernel(value, weights):
    mesh = plsc.VectorSubcoreMesh(core_axis_name="c", subcore_axis_name="s")
    nworkers = mesh.num_cores * mesh.num_subcores
    rows_per_w = B // nworkers

    vT = value.T
    wT = jnp.transpose(weights, (1, 2, 0)).reshape(N * C, B)

    run = pl.kernel(
        functools.partial(_kde_body, rows_per_w),
        out_type=jax.ShapeDtypeStruct((B * NBINS * C,), jnp.float32),
        mesh=mesh,
        compiler_params=pltpu.CompilerParams(needs_layout_passes=False),
        scratch_types=[
            pltpu.VMEM((N, BLK), jnp.float32),
            pltpu.VMEM((N * C, BLK), jnp.float32),
            pltpu.VMEM((rows_per_w * NBINS * C,), jnp.float32),
            pltpu.SemaphoreType.DMA,
            pltpu.SemaphoreType.DMA,
            pltpu.SemaphoreType.DMA,
            pltpu.SemaphoreType.DMA,
        ],
    )
    h = run(vT, wT)
    x = h.reshape(B, 2, C, 128).transpose(0, 1, 3, 2)
    return x.reshape(B, NBINS, C)
